# baseline (speedup 1.0000x reference)
import jax
import jax.numpy as jnp
from jax import lax
from jax.experimental import pallas as pl


def _psroi_kernel(ft_ref, rois_ref, out_ref):
    r = rois_ref[...]
    rsw = r[:, 1:2] * 0.125
    rsh = r[:, 2:3] * 0.125
    rew = r[:, 3:4] * 0.125
    reh = r[:, 4:5] * 0.125
    rh = reh - rsh
    rw = rew - rsw
    roih = jnp.where(rh > 0.1, rh, 0.1)
    roiw = jnp.where(rw > 0.1, rw, 0.1)
    mh = roih * (1.0 / 14.0)
    mw = roiw * (1.0 / 14.0)
    w11 = (1.0 - mw) * (1.0 - mh)
    w21 = mw * (1.0 - mh)
    w12 = (1.0 - mw) * mh
    w22 = mw * mh
    wmat = jnp.concatenate([w11, w21, w12, w22], axis=1)
    c4 = ft_ref[0, :, 0:2, 0:2].reshape(490, 4)
    out_ref[...] = lax.dot_general(
        wmat, c4, (((1,), (1,)), ((), ())),
        preferred_element_type=jnp.float32)


def kernel(ft_add_left_right, rois):
    out = pl.pallas_call(
        _psroi_kernel,
        out_shape=jax.ShapeDtypeStruct((300, 490), jnp.float32),
        grid=(1,),
        in_specs=[
            pl.BlockSpec((1, 490, 8, 34), lambda i: (0, 0, 0, 0)),
            pl.BlockSpec((300, 5), lambda i: (0, 0)),
        ],
        out_specs=pl.BlockSpec((300, 490), lambda i: (0, 0)),
    )(ft_add_left_right, rois)
    return out.reshape(300, 10, 49)


# speedup vs baseline: 1.3595x; 1.3595x over previous
import jax
import jax.numpy as jnp
from jax.experimental import pallas as pl


def _psroi_kernel(ft_ref, rois_ref, out_ref):
    r = rois_ref[...]
    rsw = r[:, 1:2] * 0.125
    rsh = r[:, 2:3] * 0.125
    rew = r[:, 3:4] * 0.125
    reh = r[:, 4:5] * 0.125
    rh = reh - rsh
    rw = rew - rsw
    roih = jnp.where(rh > 0.1, rh, 0.1)
    roiw = jnp.where(rw > 0.1, rw, 0.1)
    mh = roih * (1.0 / 14.0)
    mw = roiw * (1.0 / 14.0)
    w11 = (1.0 - mw) * (1.0 - mh)
    w21 = mw * (1.0 - mh)
    w12 = (1.0 - mw) * mh
    w22 = mw * mh
    c01 = ft_ref[:, 0:2]
    c23 = ft_ref[:, 34:36]
    c4 = jnp.concatenate([c01, c23], axis=1)
    tr = jnp.transpose(c4)
    v11 = tr[0:1, :]
    v21 = tr[1:2, :]
    v12 = tr[2:3, :]
    v22 = tr[3:4, :]
    out_ref[...] = w11 * v11 + w12 * v12 + w21 * v21 + w22 * v22


def kernel(ft_add_left_right, rois):
    ftr = ft_add_left_right.reshape(490, 1156)
    out = pl.pallas_call(
        _psroi_kernel,
        out_shape=jax.ShapeDtypeStruct((300, 490), jnp.float32),
    )(ftr, rois)
    return out.reshape(300, 10, 49)
